# baseline (speedup 1.0000x reference)
import functools

import jax
import jax.numpy as jnp
from jax import lax
from jax.experimental import pallas as pl
from jax.experimental.pallas import tpu as pltpu
from jax.experimental.pallas import tpu_sc as plsc

D_MODEL = 64
SCALE = 8.0

_NUM_CORES = 2
_NUM_SUBCORES = 16
_NW = _NUM_CORES * _NUM_SUBCORES
_C = 512
_TN = 400


def _lookup_body(seq, x_t_hbm, tab_hbm, out_hbm,
                 idx_a, idx_b, rows_a, rows_b,
                 gsem_a, gsem_b, isem_a, isem_b, ssem_a, ssem_b):
    wid = lax.axis_index("s") * _NUM_CORES + lax.axis_index("c")
    tok0 = wid * _C

    def gather_start(idx_v, rows_v, sem):
        for tcl in range(_C // 128):
            pltpu.async_copy(
                tab_hbm.at[idx_v.at[pl.ds(tcl * 128, 128)]],
                rows_v.at[tcl], sem)

    def gather_wait(idx_v, rows_v, sem):
        for tcl in range(_C // 128):
            pltpu.make_async_copy(
                tab_hbm.at[idx_v.at[pl.ds(tcl * 128, 128)]],
                rows_v.at[tcl], sem).wait()

    def scale(rows_v):
        for tcl in range(_C // 128):
            @pl.loop(0, 128, unroll=8)
            def _(t):
                for k2 in range(D_MODEL // 16):
                    sl = pl.ds(k2 * 16, 16)
                    rows_v[tcl, t, sl] = rows_v[tcl, t, sl] * SCALE

    def store_start(rows_v, s, sem):
        for dd in range(D_MODEL):
            tr, ri = dd // 8, dd % 8
            pltpu.async_copy(
                rows_v.at[:, :, dd],
                out_hbm.at[s, tr, pl.ds(wid * 4, 4), ri], sem)

    def store_wait(rows_v, s, sem):
        for dd in range(D_MODEL):
            tr, ri = dd // 8, dd % 8
            pltpu.make_async_copy(
                rows_v.at[:, :, dd],
                out_hbm.at[s, tr, pl.ds(wid * 4, 4), ri], sem).wait()

    pltpu.sync_copy(x_t_hbm.at[0, pl.ds(tok0, _C)], idx_a)
    gather_start(idx_a, rows_a, gsem_a)
    pltpu.async_copy(x_t_hbm.at[1, pl.ds(tok0, _C)], idx_b, isem_b)

    @pl.loop(0, seq, step=2)
    def pos_loop(g):
        bufs = (
            (idx_a, rows_a, gsem_a, isem_a, ssem_a,
             idx_b, rows_b, gsem_b, isem_b, ssem_b),
            (idx_b, rows_b, gsem_b, isem_b, ssem_b,
             idx_a, rows_a, gsem_a, isem_a, ssem_a),
        )
        for j, (idx_c, rows_c, gsem_c, isem_c, ssem_c,
                idx_o, rows_o, gsem_o, isem_o, ssem_o) in enumerate(bufs):
            s = g + j
            gather_wait(idx_c, rows_c, gsem_c)

            @pl.when(s + 2 < seq)
            def _():
                pltpu.async_copy(
                    x_t_hbm.at[s + 2, pl.ds(tok0, _C)], idx_c, isem_c)

            scale(rows_c)
            store_start(rows_c, s, ssem_c)

            @pl.when(s + 1 < seq)
            def _():
                @pl.when(s >= 1)
                def _():
                    store_wait(rows_o, s - 1, ssem_o)

                pltpu.make_async_copy(
                    x_t_hbm.at[s + 1, pl.ds(tok0, _C)], idx_o, isem_o).wait()
                gather_start(idx_o, rows_o, gsem_o)

    store_wait(rows_a, seq - 2, ssem_a)
    store_wait(rows_b, seq - 1, ssem_b)


def kernel(x, table):
    s0, seq = x.shape
    vocab, d = table.shape
    assert d == D_MODEL and s0 == _NW * _C and seq % 2 == 0
    x_t = x.T
    mesh = plsc.VectorSubcoreMesh(
        core_axis_name="c", subcore_axis_name="s",
        num_cores=_NUM_CORES, num_subcores=_NUM_SUBCORES)
    params = pltpu.CompilerParams(use_tc_tiling_on_sc=False, needs_layout_passes=False,
        disable_bounds_checks=True)


    out_t = pl.kernel(
        functools.partial(_lookup_body, seq),
        out_type=jax.ShapeDtypeStruct((seq, d // 8, s0 // 128, 8, 128),
                                      jnp.float32),
        mesh=mesh,
        scratch_types=[
            pltpu.VMEM((_C,), jnp.int32),
            pltpu.VMEM((_C,), jnp.int32),
            pltpu.VMEM((_C // 128, 128, D_MODEL), jnp.float32),
            pltpu.VMEM((_C // 128, 128, D_MODEL), jnp.float32),
            pltpu.SemaphoreType.DMA,
            pltpu.SemaphoreType.DMA,
            pltpu.SemaphoreType.DMA,
            pltpu.SemaphoreType.DMA,
            pltpu.SemaphoreType.DMA,
            pltpu.SemaphoreType.DMA,
        ],
        compiler_params=params,
    )(x_t, table)

    return out_t.transpose(2, 4, 0, 1, 3).reshape(s0, seq, d)


# speedup vs baseline: 393.7333x; 393.7333x over previous
import functools

import jax
import jax.numpy as jnp
from jax import lax
from jax.experimental import pallas as pl
from jax.experimental.pallas import tpu as pltpu
from jax.experimental.pallas import tpu_sc as plsc

D_MODEL = 64
SCALE = 8.0

_NUM_CORES = 2
_NUM_SUBCORES = 16
_NW = _NUM_CORES * _NUM_SUBCORES
_C = 512
_TN = 400


def _lookup_body(seq, x_t_hbm, tab_hbm, out_hbm,
                 idx_a, idx_b, rows_a, rows_b, trans,
                 gsem_a, gsem_b, isem_a, isem_b):
    wid = lax.axis_index("s") * _NUM_CORES + lax.axis_index("c")
    tok0 = wid * _C
    iota = lax.iota(jnp.int32, 16)

    def transpose_store(rows_c, s):
        @functools.partial(plsc.parallel_loop, 0, _C // 16, unroll=4)
        def _(blk):
            t_vec = iota + blk * 16
            tcl = lax.shift_right_logical(blk, 3)
            ci0 = lax.bitwise_and(blk, 7) * 16
            for dd in range(D_MODEL):
                tr, ri = dd // 8, dd % 8
                d_vec = jnp.full((16,), dd, jnp.int32)
                v = plsc.load_gather(rows_c, [t_vec, d_vec]) * SCALE
                trans[tr, tcl, ri, pl.ds(ci0, 16)] = v

        pltpu.sync_copy(trans, out_hbm.at[s, :, pl.ds(wid * 4, 4)])

    pltpu.sync_copy(x_t_hbm.at[0, pl.ds(tok0, _C)], idx_a)
    pltpu.async_copy(tab_hbm.at[idx_a], rows_a, gsem_a)
    pltpu.async_copy(x_t_hbm.at[1, pl.ds(tok0, _C)], idx_b, isem_b)

    @pl.loop(0, seq, step=2)
    def pos_loop(g):
        bufs = (
            (idx_a, rows_a, gsem_a, isem_a, idx_b, rows_b, gsem_b, isem_b),
            (idx_b, rows_b, gsem_b, isem_b, idx_a, rows_a, gsem_a, isem_a),
        )
        for j, (idx_c, rows_c, gsem_c, isem_c,
                idx_o, rows_o, gsem_o, isem_o) in enumerate(bufs):
            s = g + j
            pltpu.make_async_copy(tab_hbm.at[idx_c], rows_c, gsem_c).wait()

            @pl.when(s + 2 < seq)
            def _():
                pltpu.async_copy(
                    x_t_hbm.at[s + 2, pl.ds(tok0, _C)], idx_c, isem_c)

            @pl.when(s + 1 < seq)
            def _():
                pltpu.make_async_copy(
                    x_t_hbm.at[s + 1, pl.ds(tok0, _C)], idx_o, isem_o).wait()
                pltpu.async_copy(tab_hbm.at[idx_o], rows_o, gsem_o)

            transpose_store(rows_c, s)


def kernel(x, table):
    s0, seq = x.shape
    vocab, d = table.shape
    assert d == D_MODEL and s0 == _NW * _C and seq % 2 == 0
    x_t = x.T
    mesh = plsc.VectorSubcoreMesh(
        core_axis_name="c", subcore_axis_name="s",
        num_cores=_NUM_CORES, num_subcores=_NUM_SUBCORES)
    params = pltpu.CompilerParams(use_tc_tiling_on_sc=False, needs_layout_passes=False,
        disable_bounds_checks=True)


    out_t = pl.kernel(
        functools.partial(_lookup_body, seq),
        out_type=jax.ShapeDtypeStruct((seq, d // 8, s0 // 128, 8, 128),
                                      jnp.float32),
        mesh=mesh,
        scratch_types=[
            pltpu.VMEM((_C,), jnp.int32),
            pltpu.VMEM((_C,), jnp.int32),
            pltpu.VMEM((_C, D_MODEL), jnp.float32),
            pltpu.VMEM((_C, D_MODEL), jnp.float32),
            pltpu.VMEM((D_MODEL // 8, _C // 128, 8, 128), jnp.float32),
            pltpu.SemaphoreType.DMA,
            pltpu.SemaphoreType.DMA,
            pltpu.SemaphoreType.DMA,
            pltpu.SemaphoreType.DMA,
        ],
        compiler_params=params,
    )(x_t, table)

    return out_t.transpose(2, 4, 0, 1, 3).reshape(s0, seq, d)
